# Initial kernel scaffold; baseline (speedup 1.0000x reference)
import jax
import jax.numpy as jnp
from jax.experimental import pallas as pl


def kernel(x, edge_index, W_sd, b_sd, W_ds, b_ds, Wx0, bx0, Wx1, bx1, Wx2, bx2, Wx3, bx3):
    raise NotImplementedError("write your pallas kernel here")


# speedup vs baseline: 5.0838x; 5.0838x over previous
import functools

import jax
import jax.numpy as jnp
from jax import lax
from jax.experimental import pallas as pl
from jax.experimental.pallas import tpu as pltpu
from jax.experimental.pallas import tpu_sc as plsc

N0 = 10000
D = 128
NPAD = 10112
E0 = 320000
CHUNK = 128
NSUB = 16
NCHUNKS = 160
EPAD = NSUB * NCHUNKS * CHUNK
ROWS = NPAD // NSUB

ALPHA = 0.5
BETA = 0.5
GAMA = 0.5
C_A = (1.0 + ALPHA) * ALPHA
C_AT = (1.0 + ALPHA) * (1.0 - ALPHA)
C_IO = (1.0 + BETA) * BETA
C_OI = (1.0 + BETA) * (1.0 - BETA)
C_II = (1.0 + GAMA) * GAMA
C_OO = (1.0 + GAMA) * (1.0 - GAMA)

_MESH = dict(core_axis_name="c", subcore_axis_name="s")
_SC_PARAMS = pltpu.CompilerParams(use_tc_tiling_on_sc=False)


def _ids():
    return lax.axis_index("c"), lax.axis_index("s")


def _s1_body(row_hbm, col_hbm, z16, dr_out, dc_out, deg, idx, ones, sem):
    del sem
    core, sub = _ids()
    unit = jnp.where(
        lax.broadcasted_iota(jnp.int32, (16,), 0) == 0,
        jnp.float32(1.0), jnp.float32(0.0))

    def fill(i, c):
        ones[i, :] = unit
        return c
    lax.fori_loop(0, CHUNK, fill, 0)

    r0 = sub * ROWS
    sl = pl.ds(r0, ROWS)
    pltpu.sync_copy(z16.at[sl], deg.at[sl])
    plsc.subcore_barrier()

    def run(src_hbm):
        def step(i, c):
            base = (sub * NCHUNKS + i) * CHUNK
            pltpu.sync_copy(src_hbm.at[pl.ds(base, CHUNK)], idx)
            pltpu.sync_copy(ones, deg.at[idx], add=True)
            return c
        lax.fori_loop(0, NCHUNKS, step, 0)

    @pl.when(core == 0)
    def _():
        run(row_hbm)

    @pl.when(core == 1)
    def _():
        run(col_hbm)

    plsc.subcore_barrier()

    @pl.when(core == 0)
    def _():
        pltpu.sync_copy(deg.at[sl], dr_out.at[sl])

    @pl.when(core == 1)
    def _():
        pltpu.sync_copy(deg.at[sl], dc_out.at[sl])


_deg_sh = jax.ShapeDtypeStruct((NPAD, 16), jnp.float32)
_s1 = functools.partial(
    pl.kernel, _s1_body,
    out_type=[_deg_sh, _deg_sh],
    mesh=plsc.VectorSubcoreMesh(**_MESH),
    compiler_params=_SC_PARAMS,
    scratch_types=[
        pltpu.VMEM_SHARED((NPAD, 16), jnp.float32),
        pltpu.VMEM((CHUNK,), jnp.int32),
        pltpu.VMEM((CHUNK, 16), jnp.float32),
        pltpu.SemaphoreType.DMA,
    ],
)()


def _s2_body(row_hbm, col_hbm, dr16, dc16, z16,
             dio_out, doi_out, dii_out, doo_out,
             acc1, acc2, gidx, sidx, rows1, rows2, sem):
    core, sub = _ids()
    r0 = sub * ROWS
    sl = pl.ds(r0, ROWS)
    pltpu.sync_copy(z16.at[sl], acc1.at[sl])
    pltpu.sync_copy(z16.at[sl], acc2.at[sl])
    plsc.subcore_barrier()

    def run(g_hbm, s_hbm):
        def step(i, c):
            base = (sub * NCHUNKS + i) * CHUNK
            pltpu.sync_copy(g_hbm.at[pl.ds(base, CHUNK)], gidx)
            pltpu.sync_copy(s_hbm.at[pl.ds(base, CHUNK)], sidx)
            cp1 = pltpu.async_copy(dr16.at[gidx], rows1, sem)
            cp2 = pltpu.async_copy(dc16.at[gidx], rows2, sem)
            cp1.wait()
            cp2.wait()
            pltpu.sync_copy(rows1, acc1.at[sidx], add=True)
            pltpu.sync_copy(rows2, acc2.at[sidx], add=True)
            return c
        lax.fori_loop(0, NCHUNKS, step, 0)

    @pl.when(core == 0)
    def _():
        run(col_hbm, row_hbm)

    @pl.when(core == 1)
    def _():
        run(row_hbm, col_hbm)

    plsc.subcore_barrier()

    @pl.when(core == 0)
    def _():
        pltpu.sync_copy(acc1.at[sl], dii_out.at[sl])
        pltpu.sync_copy(acc2.at[sl], dio_out.at[sl])

    @pl.when(core == 1)
    def _():
        pltpu.sync_copy(acc1.at[sl], doi_out.at[sl])
        pltpu.sync_copy(acc2.at[sl], doo_out.at[sl])


_s2 = functools.partial(
    pl.kernel, _s2_body,
    out_type=[_deg_sh, _deg_sh, _deg_sh, _deg_sh],
    mesh=plsc.VectorSubcoreMesh(**_MESH),
    compiler_params=_SC_PARAMS,
    scratch_types=[
        pltpu.VMEM_SHARED((NPAD, 16), jnp.float32),
        pltpu.VMEM_SHARED((NPAD, 16), jnp.float32),
        pltpu.VMEM((CHUNK,), jnp.int32),
        pltpu.VMEM((CHUNK,), jnp.int32),
        pltpu.VMEM((CHUNK, 16), jnp.float32),
        pltpu.VMEM((CHUNK, 16), jnp.float32),
        pltpu.SemaphoreType.DMA,
    ],
)()


def _rs(d):
    return jnp.where(d > 0, lax.rsqrt(d), 0.0)


def _t1_body(x_ref, dr_ref, dc_ref, dio_ref, doi_ref, dii_ref, doo_ref,
             y1_ref, y2_ref, y3_ref, y4_ref, y5_ref, y6_ref):
    xb = x_ref[...]
    y1_ref[...] = _rs(dc_ref[:, 0:1]) * xb
    y2_ref[...] = _rs(dr_ref[:, 0:1]) * xb
    y3_ref[...] = _rs(dio_ref[:, 0:1]) * xb
    y4_ref[...] = _rs(doi_ref[:, 0:1]) * xb
    y5_ref[...] = _rs(doo_ref[:, 0:1]) * xb
    y6_ref[...] = _rs(dii_ref[:, 0:1]) * xb


_BT = 632


def _t1(xp, dr16, dc16, dio16, doi16, dii16, doo16):
    y_sh = jax.ShapeDtypeStruct((NPAD, D), jnp.float32)
    x_spec = pl.BlockSpec((_BT, D), lambda i: (i, 0))
    d_spec = pl.BlockSpec((_BT, 16), lambda i: (i, 0))
    return pl.pallas_call(
        _t1_body,
        grid=(NPAD // _BT,),
        in_specs=[x_spec] + [d_spec] * 6,
        out_specs=[x_spec] * 6,
        out_shape=[y_sh] * 6,
    )(xp, dr16, dc16, dio16, doi16, dii16, doo16)


def _agg_pass(y_hbm, g_src, s_src, out_hbm, z128, acc, gidx, sidx, rows, sem,
              sub):
    r0 = sub * ROWS
    sl = pl.ds(r0, ROWS)
    pltpu.sync_copy(z128.at[sl], acc.at[sl])
    plsc.subcore_barrier()

    def step(i, c):
        base = (sub * NCHUNKS + i) * CHUNK
        pltpu.sync_copy(g_src.at[pl.ds(base, CHUNK)], gidx)
        pltpu.sync_copy(s_src.at[pl.ds(base, CHUNK)], sidx)
        pltpu.async_copy(y_hbm.at[gidx], rows, sem).wait()
        pltpu.sync_copy(rows, acc.at[sidx], add=True)
        return c
    lax.fori_loop(0, NCHUNKS, step, 0)
    plsc.subcore_barrier()
    pltpu.sync_copy(acc.at[sl], out_hbm.at[sl])
    plsc.subcore_barrier()


def _s3_body(row_hbm, col_hbm, y1, y2, y3, y4, y5, y6, z128,
             preA, preT, n_oi, n_io, n_ii, n_oo,
             acc, gidx, sidx, rows, sem):
    core, sub = _ids()

    @pl.when(core == 0)
    def _():
        _agg_pass(y1, col_hbm, row_hbm, preA, z128, acc, gidx, sidx, rows,
                  sem, sub)
        _agg_pass(y4, col_hbm, row_hbm, n_oi, z128, acc, gidx, sidx, rows,
                  sem, sub)
        _agg_pass(y5, col_hbm, row_hbm, n_ii, z128, acc, gidx, sidx, rows,
                  sem, sub)

    @pl.when(core == 1)
    def _():
        _agg_pass(y2, row_hbm, col_hbm, preT, z128, acc, gidx, sidx, rows,
                  sem, sub)
        _agg_pass(y3, row_hbm, col_hbm, n_io, z128, acc, gidx, sidx, rows,
                  sem, sub)
        _agg_pass(y6, row_hbm, col_hbm, n_oo, z128, acc, gidx, sidx, rows,
                  sem, sub)


_mat_sh = jax.ShapeDtypeStruct((NPAD, D), jnp.float32)
_agg_scratch = [
    pltpu.VMEM_SHARED((NPAD, D), jnp.float32),
    pltpu.VMEM((CHUNK,), jnp.int32),
    pltpu.VMEM((CHUNK,), jnp.int32),
    pltpu.VMEM((CHUNK, D), jnp.float32),
    pltpu.SemaphoreType.DMA,
]

_s3 = functools.partial(
    pl.kernel, _s3_body,
    out_type=[_mat_sh] * 6,
    mesh=plsc.VectorSubcoreMesh(**_MESH),
    compiler_params=_SC_PARAMS,
    scratch_types=_agg_scratch,
)()


def _s4_body(row_hbm, col_hbm, n_io, n_ii, n_oi, n_oo, z128,
             pio, pii, poi, poo,
             acc, gidx, sidx, rows, sem):
    core, sub = _ids()

    @pl.when(core == 0)
    def _():
        _agg_pass(n_io, col_hbm, row_hbm, pio, z128, acc, gidx, sidx, rows,
                  sem, sub)
        _agg_pass(n_ii, col_hbm, row_hbm, pii, z128, acc, gidx, sidx, rows,
                  sem, sub)

    @pl.when(core == 1)
    def _():
        _agg_pass(n_oi, row_hbm, col_hbm, poi, z128, acc, gidx, sidx, rows,
                  sem, sub)
        _agg_pass(n_oo, row_hbm, col_hbm, poo, z128, acc, gidx, sidx, rows,
                  sem, sub)


_s4 = functools.partial(
    pl.kernel, _s4_body,
    out_type=[_mat_sh] * 4,
    mesh=plsc.VectorSubcoreMesh(**_MESH),
    compiler_params=_SC_PARAMS,
    scratch_types=_agg_scratch,
)()


def _t2_body(pa, pt, pio, poi, pii, poo,
             dr_ref, dc_ref, dio_ref, doi_ref, dii_ref, doo_ref,
             wsd, wds, w0, w1, w2, w3,
             bsd, bds, b0, b1, b2, b3, o_ref):
    def term(pre_ref, d_ref, coeff, w_ref):
        h = (coeff * _rs(d_ref[:, 0:1])) * pre_ref[...]
        return lax.dot_general(h, w_ref[...], (((1,), (1,)), ((), ())),
                               preferred_element_type=jnp.float32)

    acc = term(pa, dr_ref, C_A, wsd)
    acc += term(pt, dc_ref, C_AT, wds)
    acc += term(pio, dio_ref, C_IO, w0)
    acc += term(poi, doi_ref, C_OI, w1)
    acc += term(pii, dii_ref, C_II, w2)
    acc += term(poo, doo_ref, C_OO, w3)
    bsum = (C_A * bsd[...] + C_AT * bds[...] + C_IO * b0[...]
            + C_OI * b1[...] + C_II * b2[...] + C_OO * b3[...])
    o_ref[...] = acc + bsum


def _t2(pres, degs, ws, bs):
    pre_spec = pl.BlockSpec((_BT, D), lambda i: (i, 0))
    deg_spec = pl.BlockSpec((_BT, 16), lambda i: (i, 0))
    w_spec = pl.BlockSpec((D, D), lambda i: (0, 0))
    b_spec = pl.BlockSpec((1, D), lambda i: (0, 0))
    return pl.pallas_call(
        _t2_body,
        grid=(NPAD // _BT,),
        in_specs=[pre_spec] * 6 + [deg_spec] * 6 + [w_spec] * 6 + [b_spec] * 6,
        out_specs=pl.BlockSpec((_BT, D), lambda i: (i, 0)),
        out_shape=jax.ShapeDtypeStruct((NPAD, D), jnp.float32),
    )(*pres, *degs, *ws, *bs)


def kernel(x, edge_index, W_sd, b_sd, W_ds, b_ds,
           Wx0, bx0, Wx1, bx1, Wx2, bx2, Wx3, bx3):
    row = edge_index[0]
    col = edge_index[1]
    pad = EPAD - row.shape[0]
    sink = jnp.full((pad,), N0, jnp.int32)
    rowp = jnp.concatenate([row.astype(jnp.int32), sink])
    colp = jnp.concatenate([col.astype(jnp.int32), sink])
    xp = jnp.zeros((NPAD, D), jnp.float32).at[:N0].set(x)

    z16 = jnp.zeros((NPAD, 16), jnp.float32)
    z128 = jnp.zeros((NPAD, D), jnp.float32)

    dr16, dc16 = _s1(rowp, colp, z16)
    dio16, doi16, dii16, doo16 = _s2(rowp, colp, dr16, dc16, z16)
    y1, y2, y3, y4, y5, y6 = _t1(xp, dr16, dc16, dio16, doi16, dii16, doo16)
    preA, preT, n_oi, n_io, n_ii, n_oo = _s3(
        rowp, colp, y1, y2, y3, y4, y5, y6, z128)
    pio, pii, poi, poo = _s4(rowp, colp, n_io, n_ii, n_oi, n_oo, z128)
    out = _t2((preA, preT, pio, poi, pii, poo),
              (dr16, dc16, dio16, doi16, dii16, doo16),
              (W_sd, W_ds, Wx0, Wx1, Wx2, Wx3),
              (b_sd.reshape(1, D), b_ds.reshape(1, D), bx0.reshape(1, D),
               bx1.reshape(1, D), bx2.reshape(1, D), bx3.reshape(1, D)))
    return out[:N0]


# speedup vs baseline: 6.5724x; 1.2928x over previous
import functools

import jax
import jax.numpy as jnp
from jax import lax
from jax.experimental import pallas as pl
from jax.experimental.pallas import tpu as pltpu
from jax.experimental.pallas import tpu_sc as plsc

N0 = 10000
D = 128
NPAD = 10112
E0 = 320000
CHUNK = 128
NSUB = 16
NCHUNKS = 160
EPAD = NSUB * NCHUNKS * CHUNK
ROWS = NPAD // NSUB

ALPHA = 0.5
BETA = 0.5
GAMA = 0.5
C_A = (1.0 + ALPHA) * ALPHA
C_AT = (1.0 + ALPHA) * (1.0 - ALPHA)
C_IO = (1.0 + BETA) * BETA
C_OI = (1.0 + BETA) * (1.0 - BETA)
C_II = (1.0 + GAMA) * GAMA
C_OO = (1.0 + GAMA) * (1.0 - GAMA)

_MESH = dict(core_axis_name="c", subcore_axis_name="s")
_SC_PARAMS = pltpu.CompilerParams(use_tc_tiling_on_sc=False)


def _ids():
    return lax.axis_index("c"), lax.axis_index("s")


def _s1_body(row_hbm, col_hbm, z16, dr_out, dc_out, deg, idx, ones, sem):
    del sem
    core, sub = _ids()
    unit = jnp.where(
        lax.broadcasted_iota(jnp.int32, (16,), 0) == 0,
        jnp.float32(1.0), jnp.float32(0.0))

    def fill(i, c):
        ones[i, :] = unit
        return c
    lax.fori_loop(0, CHUNK, fill, 0)

    r0 = sub * ROWS
    sl = pl.ds(r0, ROWS)
    pltpu.sync_copy(z16.at[sl], deg.at[sl])
    plsc.subcore_barrier()

    def run(src_hbm):
        def step(i, c):
            base = (sub * NCHUNKS + i) * CHUNK
            pltpu.sync_copy(src_hbm.at[pl.ds(base, CHUNK)], idx)
            pltpu.sync_copy(ones, deg.at[idx], add=True)
            return c
        lax.fori_loop(0, NCHUNKS, step, 0)

    @pl.when(core == 0)
    def _():
        run(row_hbm)

    @pl.when(core == 1)
    def _():
        run(col_hbm)

    plsc.subcore_barrier()

    @pl.when(core == 0)
    def _():
        pltpu.sync_copy(deg.at[sl], dr_out.at[sl])

    @pl.when(core == 1)
    def _():
        pltpu.sync_copy(deg.at[sl], dc_out.at[sl])


_deg_sh = jax.ShapeDtypeStruct((NPAD, 16), jnp.float32)
_s1 = functools.partial(
    pl.kernel, _s1_body,
    out_type=[_deg_sh, _deg_sh],
    mesh=plsc.VectorSubcoreMesh(**_MESH),
    compiler_params=_SC_PARAMS,
    scratch_types=[
        pltpu.VMEM_SHARED((NPAD, 16), jnp.float32),
        pltpu.VMEM((CHUNK,), jnp.int32),
        pltpu.VMEM((CHUNK, 16), jnp.float32),
        pltpu.SemaphoreType.DMA,
    ],
)()


def _s2_body(row_hbm, col_hbm, dr16, dc16, z16,
             dio_out, doi_out, dii_out, doo_out,
             acc1, acc2, gidx, sidx, rows1, rows2, sem):
    core, sub = _ids()
    r0 = sub * ROWS
    sl = pl.ds(r0, ROWS)
    pltpu.sync_copy(z16.at[sl], acc1.at[sl])
    pltpu.sync_copy(z16.at[sl], acc2.at[sl])
    plsc.subcore_barrier()

    def run(g_hbm, s_hbm):
        def step(i, c):
            base = (sub * NCHUNKS + i) * CHUNK
            pltpu.sync_copy(g_hbm.at[pl.ds(base, CHUNK)], gidx)
            pltpu.sync_copy(s_hbm.at[pl.ds(base, CHUNK)], sidx)
            cp1 = pltpu.async_copy(dr16.at[gidx], rows1, sem)
            cp2 = pltpu.async_copy(dc16.at[gidx], rows2, sem)
            cp1.wait()
            cp2.wait()
            pltpu.sync_copy(rows1, acc1.at[sidx], add=True)
            pltpu.sync_copy(rows2, acc2.at[sidx], add=True)
            return c
        lax.fori_loop(0, NCHUNKS, step, 0)

    @pl.when(core == 0)
    def _():
        run(col_hbm, row_hbm)

    @pl.when(core == 1)
    def _():
        run(row_hbm, col_hbm)

    plsc.subcore_barrier()

    @pl.when(core == 0)
    def _():
        pltpu.sync_copy(acc1.at[sl], dii_out.at[sl])
        pltpu.sync_copy(acc2.at[sl], dio_out.at[sl])

    @pl.when(core == 1)
    def _():
        pltpu.sync_copy(acc1.at[sl], doi_out.at[sl])
        pltpu.sync_copy(acc2.at[sl], doo_out.at[sl])


_s2 = functools.partial(
    pl.kernel, _s2_body,
    out_type=[_deg_sh, _deg_sh, _deg_sh, _deg_sh],
    mesh=plsc.VectorSubcoreMesh(**_MESH),
    compiler_params=_SC_PARAMS,
    scratch_types=[
        pltpu.VMEM_SHARED((NPAD, 16), jnp.float32),
        pltpu.VMEM_SHARED((NPAD, 16), jnp.float32),
        pltpu.VMEM((CHUNK,), jnp.int32),
        pltpu.VMEM((CHUNK,), jnp.int32),
        pltpu.VMEM((CHUNK, 16), jnp.float32),
        pltpu.VMEM((CHUNK, 16), jnp.float32),
        pltpu.SemaphoreType.DMA,
    ],
)()


def _rs(d):
    return jnp.where(d > 0, lax.rsqrt(d), 0.0)


def _t1_body(x_ref, dr_ref, dc_ref, dio_ref, doi_ref, dii_ref, doo_ref,
             y1_ref, y2_ref, y3_ref, y4_ref, y5_ref, y6_ref):
    xb = x_ref[...]
    y1_ref[...] = _rs(dc_ref[:, 0:1]) * xb
    y2_ref[...] = _rs(dr_ref[:, 0:1]) * xb
    y3_ref[...] = _rs(dio_ref[:, 0:1]) * xb
    y4_ref[...] = _rs(doi_ref[:, 0:1]) * xb
    y5_ref[...] = _rs(doo_ref[:, 0:1]) * xb
    y6_ref[...] = _rs(dii_ref[:, 0:1]) * xb


_BT = 632


def _t1(xp, dr16, dc16, dio16, doi16, dii16, doo16):
    y_sh = jax.ShapeDtypeStruct((NPAD, D), jnp.float32)
    x_spec = pl.BlockSpec((_BT, D), lambda i: (i, 0))
    d_spec = pl.BlockSpec((_BT, 16), lambda i: (i, 0))
    return pl.pallas_call(
        _t1_body,
        grid=(NPAD // _BT,),
        in_specs=[x_spec] + [d_spec] * 6,
        out_specs=[x_spec] * 6,
        out_shape=[y_sh] * 6,
    )(xp, dr16, dc16, dio16, doi16, dii16, doo16)


def _agg_pass(y_hbm, g_src, s_src, out_hbm, z128, acc, gidx, sidx, rows,
              gsem, ssem, sub):
    r0 = sub * ROWS
    sl = pl.ds(r0, ROWS)
    pltpu.sync_copy(z128.at[sl], acc.at[sl])
    plsc.subcore_barrier()

    def start_gather(i, b):
        base = (sub * NCHUNKS + i) * CHUNK
        pltpu.sync_copy(g_src.at[pl.ds(base, CHUNK)], gidx.at[b])
        pltpu.sync_copy(s_src.at[pl.ds(base, CHUNK)], sidx.at[b])
        pltpu.async_copy(y_hbm.at[gidx.at[b]], rows.at[b], gsem[b])

    def wait_gather(b):
        pltpu.make_async_copy(y_hbm.at[gidx.at[b]], rows.at[b],
                              gsem[b]).wait()

    def start_scatter(b):
        pltpu.async_copy(rows.at[b], acc.at[sidx.at[b]], ssem[b], add=True)

    def wait_scatter(b):
        pltpu.make_async_copy(rows.at[b], acc.at[sidx.at[b]], ssem[b]).wait()

    start_gather(0, 0)
    start_gather(1, 1)
    wait_gather(0)
    start_scatter(0)

    def body(g, c):
        wait_scatter(0)
        start_gather(2 * g + 2, 0)
        wait_gather(1)
        start_scatter(1)
        wait_scatter(1)
        start_gather(2 * g + 3, 1)
        wait_gather(0)
        start_scatter(0)
        return c
    lax.fori_loop(0, (NCHUNKS - 2) // 2, body, 0)
    wait_gather(1)
    start_scatter(1)
    wait_scatter(0)
    wait_scatter(1)
    plsc.subcore_barrier()
    pltpu.sync_copy(acc.at[sl], out_hbm.at[sl])
    plsc.subcore_barrier()


def _s3_body(row_hbm, col_hbm, y1, y2, y3, y4, y5, y6, z128,
             preA, preT, n_oi, n_io, n_ii, n_oo,
             acc, gidx, sidx, rows, gs0, gs1, ss0, ss1):
    core, sub = _ids()
    gsem = (gs0, gs1)
    ssem = (ss0, ss1)

    @pl.when(core == 0)
    def _():
        _agg_pass(y1, col_hbm, row_hbm, preA, z128, acc, gidx, sidx, rows,
                  gsem, ssem, sub)
        _agg_pass(y4, col_hbm, row_hbm, n_oi, z128, acc, gidx, sidx, rows,
                  gsem, ssem, sub)
        _agg_pass(y5, col_hbm, row_hbm, n_ii, z128, acc, gidx, sidx, rows,
                  gsem, ssem, sub)

    @pl.when(core == 1)
    def _():
        _agg_pass(y2, row_hbm, col_hbm, preT, z128, acc, gidx, sidx, rows,
                  gsem, ssem, sub)
        _agg_pass(y3, row_hbm, col_hbm, n_io, z128, acc, gidx, sidx, rows,
                  gsem, ssem, sub)
        _agg_pass(y6, row_hbm, col_hbm, n_oo, z128, acc, gidx, sidx, rows,
                  gsem, ssem, sub)


_mat_sh = jax.ShapeDtypeStruct((NPAD, D), jnp.float32)
_agg_scratch = [
    pltpu.VMEM_SHARED((NPAD, D), jnp.float32),
    pltpu.VMEM((2, CHUNK), jnp.int32),
    pltpu.VMEM((2, CHUNK), jnp.int32),
    pltpu.VMEM((2, CHUNK, D), jnp.float32),
    pltpu.SemaphoreType.DMA,
    pltpu.SemaphoreType.DMA,
    pltpu.SemaphoreType.DMA,
    pltpu.SemaphoreType.DMA,
]

_s3 = functools.partial(
    pl.kernel, _s3_body,
    out_type=[_mat_sh] * 6,
    mesh=plsc.VectorSubcoreMesh(**_MESH),
    compiler_params=_SC_PARAMS,
    scratch_types=_agg_scratch,
)()


def _s4_body(row_hbm, col_hbm, n_io, n_ii, n_oi, n_oo, z128,
             pio, pii, poi, poo,
             acc, gidx, sidx, rows, gs0, gs1, ss0, ss1):
    core, sub = _ids()
    gsem = (gs0, gs1)
    ssem = (ss0, ss1)

    @pl.when(core == 0)
    def _():
        _agg_pass(n_io, col_hbm, row_hbm, pio, z128, acc, gidx, sidx, rows,
                  gsem, ssem, sub)
        _agg_pass(n_ii, col_hbm, row_hbm, pii, z128, acc, gidx, sidx, rows,
                  gsem, ssem, sub)

    @pl.when(core == 1)
    def _():
        _agg_pass(n_oi, row_hbm, col_hbm, poi, z128, acc, gidx, sidx, rows,
                  gsem, ssem, sub)
        _agg_pass(n_oo, row_hbm, col_hbm, poo, z128, acc, gidx, sidx, rows,
                  gsem, ssem, sub)


_s4 = functools.partial(
    pl.kernel, _s4_body,
    out_type=[_mat_sh] * 4,
    mesh=plsc.VectorSubcoreMesh(**_MESH),
    compiler_params=_SC_PARAMS,
    scratch_types=_agg_scratch,
)()


def _t2_body(pa, pt, pio, poi, pii, poo,
             dr_ref, dc_ref, dio_ref, doi_ref, dii_ref, doo_ref,
             wsd, wds, w0, w1, w2, w3,
             bsd, bds, b0, b1, b2, b3, o_ref):
    def term(pre_ref, d_ref, coeff, w_ref):
        h = (coeff * _rs(d_ref[:, 0:1])) * pre_ref[...]
        return lax.dot_general(h, w_ref[...], (((1,), (1,)), ((), ())),
                               preferred_element_type=jnp.float32)

    acc = term(pa, dr_ref, C_A, wsd)
    acc += term(pt, dc_ref, C_AT, wds)
    acc += term(pio, dio_ref, C_IO, w0)
    acc += term(poi, doi_ref, C_OI, w1)
    acc += term(pii, dii_ref, C_II, w2)
    acc += term(poo, doo_ref, C_OO, w3)
    bsum = (C_A * bsd[...] + C_AT * bds[...] + C_IO * b0[...]
            + C_OI * b1[...] + C_II * b2[...] + C_OO * b3[...])
    o_ref[...] = acc + bsum


def _t2(pres, degs, ws, bs):
    pre_spec = pl.BlockSpec((_BT, D), lambda i: (i, 0))
    deg_spec = pl.BlockSpec((_BT, 16), lambda i: (i, 0))
    w_spec = pl.BlockSpec((D, D), lambda i: (0, 0))
    b_spec = pl.BlockSpec((1, D), lambda i: (0, 0))
    return pl.pallas_call(
        _t2_body,
        grid=(NPAD // _BT,),
        in_specs=[pre_spec] * 6 + [deg_spec] * 6 + [w_spec] * 6 + [b_spec] * 6,
        out_specs=pl.BlockSpec((_BT, D), lambda i: (i, 0)),
        out_shape=jax.ShapeDtypeStruct((NPAD, D), jnp.float32),
    )(*pres, *degs, *ws, *bs)


def kernel(x, edge_index, W_sd, b_sd, W_ds, b_ds,
           Wx0, bx0, Wx1, bx1, Wx2, bx2, Wx3, bx3):
    row = edge_index[0]
    col = edge_index[1]
    pad = EPAD - row.shape[0]
    sink = jnp.full((pad,), N0, jnp.int32)
    rowp = jnp.concatenate([row.astype(jnp.int32), sink])
    colp = jnp.concatenate([col.astype(jnp.int32), sink])
    xp = jnp.zeros((NPAD, D), jnp.float32).at[:N0].set(x)

    z16 = jnp.zeros((NPAD, 16), jnp.float32)
    z128 = jnp.zeros((NPAD, D), jnp.float32)

    dr16, dc16 = _s1(rowp, colp, z16)
    dio16, doi16, dii16, doo16 = _s2(rowp, colp, dr16, dc16, z16)
    y1, y2, y3, y4, y5, y6 = _t1(xp, dr16, dc16, dio16, doi16, dii16, doo16)
    preA, preT, n_oi, n_io, n_ii, n_oo = _s3(
        rowp, colp, y1, y2, y3, y4, y5, y6, z128)
    pio, pii, poi, poo = _s4(rowp, colp, n_io, n_ii, n_oi, n_oo, z128)
    out = _t2((preA, preT, pio, poi, pii, poo),
              (dr16, dc16, dio16, doi16, dii16, doo16),
              (W_sd, W_ds, Wx0, Wx1, Wx2, Wx3),
              (b_sd.reshape(1, D), b_ds.reshape(1, D), bx0.reshape(1, D),
               bx1.reshape(1, D), bx2.reshape(1, D), bx3.reshape(1, D)))
    return out[:N0]


# speedup vs baseline: 6.8762x; 1.0462x over previous
import functools

import jax
import jax.numpy as jnp
from jax import lax
from jax.experimental import pallas as pl
from jax.experimental.pallas import tpu as pltpu
from jax.experimental.pallas import tpu_sc as plsc

N0 = 10000
D = 128
NPAD = 10112
E0 = 320000
CHUNK = 128
NSUB = 16
NCHUNKS = 160
EPAD = NSUB * NCHUNKS * CHUNK
ROWS = NPAD // NSUB
G = 8
NGRP = NCHUNKS // G

ALPHA = 0.5
BETA = 0.5
GAMA = 0.5
C_A = (1.0 + ALPHA) * ALPHA
C_AT = (1.0 + ALPHA) * (1.0 - ALPHA)
C_IO = (1.0 + BETA) * BETA
C_OI = (1.0 + BETA) * (1.0 - BETA)
C_II = (1.0 + GAMA) * GAMA
C_OO = (1.0 + GAMA) * (1.0 - GAMA)

_MESH = dict(core_axis_name="c", subcore_axis_name="s")
_SC_PARAMS = pltpu.CompilerParams(use_tc_tiling_on_sc=False)


def _ids():
    return lax.axis_index("c"), lax.axis_index("s")


def _s1_body(row_hbm, col_hbm, z16, dr_out, dc_out, deg, idx, ones, sem):
    del sem
    core, sub = _ids()
    unit = jnp.where(
        lax.broadcasted_iota(jnp.int32, (16,), 0) == 0,
        jnp.float32(1.0), jnp.float32(0.0))

    def fill(i, c):
        ones[i, :] = unit
        return c
    lax.fori_loop(0, CHUNK, fill, 0)

    r0 = sub * ROWS
    sl = pl.ds(r0, ROWS)
    pltpu.sync_copy(z16.at[sl], deg.at[sl])
    plsc.subcore_barrier()

    def run(src_hbm):
        def step(i, c):
            base = (sub * NCHUNKS + i) * CHUNK
            pltpu.sync_copy(src_hbm.at[pl.ds(base, CHUNK)], idx)
            pltpu.sync_copy(ones, deg.at[idx], add=True)
            return c
        lax.fori_loop(0, NCHUNKS, step, 0)

    @pl.when(core == 0)
    def _():
        run(row_hbm)

    @pl.when(core == 1)
    def _():
        run(col_hbm)

    plsc.subcore_barrier()

    @pl.when(core == 0)
    def _():
        pltpu.sync_copy(deg.at[sl], dr_out.at[sl])

    @pl.when(core == 1)
    def _():
        pltpu.sync_copy(deg.at[sl], dc_out.at[sl])


_deg_sh = jax.ShapeDtypeStruct((NPAD, 16), jnp.float32)
_s1 = functools.partial(
    pl.kernel, _s1_body,
    out_type=[_deg_sh, _deg_sh],
    mesh=plsc.VectorSubcoreMesh(**_MESH),
    compiler_params=_SC_PARAMS,
    scratch_types=[
        pltpu.VMEM_SHARED((NPAD, 16), jnp.float32),
        pltpu.VMEM((CHUNK,), jnp.int32),
        pltpu.VMEM((CHUNK, 16), jnp.float32),
        pltpu.SemaphoreType.DMA,
    ],
)()


def _s2_body(row_hbm, col_hbm, d1cat, z32, d2a_out, d2t_out,
             acc, gidx, sidx, rows, gs0, gs1, ss0, ss1):
    core, sub = _ids()
    gsem = (gs0, gs1)
    ssem = (ss0, ss1)

    @pl.when(core == 0)
    def _():
        _agg_pass(d1cat, col_hbm, row_hbm, d2a_out, z32, acc, gidx, sidx,
                  rows, gsem, ssem, sub)

    @pl.when(core == 1)
    def _():
        _agg_pass(d1cat, row_hbm, col_hbm, d2t_out, z32, acc, gidx, sidx,
                  rows, gsem, ssem, sub)


_deg2_sh = jax.ShapeDtypeStruct((NPAD, 32), jnp.float32)
_s2 = functools.partial(
    pl.kernel, _s2_body,
    out_type=[_deg2_sh, _deg2_sh],
    mesh=plsc.VectorSubcoreMesh(**_MESH),
    compiler_params=_SC_PARAMS,
    scratch_types=[
        pltpu.VMEM_SHARED((NPAD, 32), jnp.float32),
        pltpu.VMEM((2, CHUNK), jnp.int32),
        pltpu.VMEM((2, CHUNK), jnp.int32),
        pltpu.VMEM((2, CHUNK, 32), jnp.float32),
        pltpu.SemaphoreType.DMA,
        pltpu.SemaphoreType.DMA,
        pltpu.SemaphoreType.DMA,
        pltpu.SemaphoreType.DMA,
    ],
)()


def _rs(d):
    return jnp.where(d > 0, lax.rsqrt(d), 0.0)


def _t1_body(x_ref, dr_ref, dc_ref, d2a_ref, d2t_ref,
             y1_ref, y2_ref, y3_ref, y4_ref, y5_ref, y6_ref):
    xb = x_ref[...]
    d2a = d2a_ref[...]
    d2t = d2t_ref[...]
    y1_ref[...] = _rs(dc_ref[:, 0:1]) * xb
    y2_ref[...] = _rs(dr_ref[:, 0:1]) * xb
    y3_ref[...] = _rs(d2a[:, 16:17]) * xb
    y4_ref[...] = _rs(d2t[:, 0:1]) * xb
    y5_ref[...] = _rs(d2t[:, 16:17]) * xb
    y6_ref[...] = _rs(d2a[:, 0:1]) * xb


_BT = 632


def _t1(xp, dr16, dc16, d2a, d2t):
    y_sh = jax.ShapeDtypeStruct((NPAD, D), jnp.float32)
    x_spec = pl.BlockSpec((_BT, D), lambda i: (i, 0))
    d_spec = pl.BlockSpec((_BT, 16), lambda i: (i, 0))
    d2_spec = pl.BlockSpec((_BT, 32), lambda i: (i, 0))
    return pl.pallas_call(
        _t1_body,
        grid=(NPAD // _BT,),
        in_specs=[x_spec, d_spec, d_spec, d2_spec, d2_spec],
        out_specs=[x_spec] * 6,
        out_shape=[y_sh] * 6,
    )(xp, dr16, dc16, d2a, d2t)


def _agg_pass(y_hbm, g_src, s_src, out_hbm, zeros, acc, gidx, sidx, rows,
              gsem, ssem, sub):
    r0 = sub * ROWS
    sl = pl.ds(r0, ROWS)
    pltpu.sync_copy(zeros.at[sl], acc.at[sl])
    plsc.subcore_barrier()

    def start_gather(i, b):
        base = (sub * NCHUNKS + i) * CHUNK
        pltpu.sync_copy(g_src.at[pl.ds(base, CHUNK)], gidx.at[b])
        pltpu.sync_copy(s_src.at[pl.ds(base, CHUNK)], sidx.at[b])
        pltpu.async_copy(y_hbm.at[gidx.at[b]], rows.at[b], gsem[b])

    def wait_gather(b):
        pltpu.make_async_copy(y_hbm.at[gidx.at[b]], rows.at[b],
                              gsem[b]).wait()

    def start_scatter(b):
        pltpu.async_copy(rows.at[b], acc.at[sidx.at[b]], ssem[b], add=True)

    def wait_scatter(b):
        pltpu.make_async_copy(rows.at[b], acc.at[sidx.at[b]], ssem[b]).wait()

    start_gather(0, 0)
    start_gather(1, 1)
    wait_gather(0)
    start_scatter(0)

    def body(g, c):
        wait_scatter(0)
        start_gather(2 * g + 2, 0)
        wait_gather(1)
        start_scatter(1)
        wait_scatter(1)
        start_gather(2 * g + 3, 1)
        wait_gather(0)
        start_scatter(0)
        return c
    lax.fori_loop(0, (NCHUNKS - 2) // 2, body, 0)
    wait_gather(1)
    start_scatter(1)
    wait_scatter(0)
    wait_scatter(1)
    plsc.subcore_barrier()
    pltpu.sync_copy(acc.at[sl], out_hbm.at[sl])
    plsc.subcore_barrier()


def _s3_body(row_hbm, col_hbm, y1, y2, y3, y4, y5, y6, z128,
             preA, preT, n_oi, n_io, n_ii, n_oo,
             acc, gidx, sidx, rows, gs0, gs1, ss0, ss1):
    core, sub = _ids()
    gsem = (gs0, gs1)
    ssem = (ss0, ss1)

    @pl.when(core == 0)
    def _():
        _agg_pass(y1, col_hbm, row_hbm, preA, z128, acc, gidx, sidx, rows,
                  gsem, ssem, sub)
        _agg_pass(y4, col_hbm, row_hbm, n_oi, z128, acc, gidx, sidx, rows,
                  gsem, ssem, sub)
        _agg_pass(y5, col_hbm, row_hbm, n_ii, z128, acc, gidx, sidx, rows,
                  gsem, ssem, sub)

    @pl.when(core == 1)
    def _():
        _agg_pass(y2, row_hbm, col_hbm, preT, z128, acc, gidx, sidx, rows,
                  gsem, ssem, sub)
        _agg_pass(y3, row_hbm, col_hbm, n_io, z128, acc, gidx, sidx, rows,
                  gsem, ssem, sub)
        _agg_pass(y6, row_hbm, col_hbm, n_oo, z128, acc, gidx, sidx, rows,
                  gsem, ssem, sub)


_mat_sh = jax.ShapeDtypeStruct((NPAD, D), jnp.float32)
_agg_scratch = [
    pltpu.VMEM_SHARED((NPAD, D), jnp.float32),
    pltpu.VMEM((2, CHUNK), jnp.int32),
    pltpu.VMEM((2, CHUNK), jnp.int32),
    pltpu.VMEM((2, CHUNK, D), jnp.float32),
    pltpu.SemaphoreType.DMA,
    pltpu.SemaphoreType.DMA,
    pltpu.SemaphoreType.DMA,
    pltpu.SemaphoreType.DMA,
]

_s3 = functools.partial(
    pl.kernel, _s3_body,
    out_type=[_mat_sh] * 6,
    mesh=plsc.VectorSubcoreMesh(**_MESH),
    compiler_params=_SC_PARAMS,
    scratch_types=_agg_scratch,
)()


def _s4_body(row_hbm, col_hbm, n_io, n_ii, n_oi, n_oo, z128,
             pio, pii, poi, poo,
             acc, gidx, sidx, rows, gs0, gs1, ss0, ss1):
    core, sub = _ids()
    gsem = (gs0, gs1)
    ssem = (ss0, ss1)

    @pl.when(core == 0)
    def _():
        _agg_pass(n_io, col_hbm, row_hbm, pio, z128, acc, gidx, sidx, rows,
                  gsem, ssem, sub)
        _agg_pass(n_ii, col_hbm, row_hbm, pii, z128, acc, gidx, sidx, rows,
                  gsem, ssem, sub)

    @pl.when(core == 1)
    def _():
        _agg_pass(n_oi, row_hbm, col_hbm, poi, z128, acc, gidx, sidx, rows,
                  gsem, ssem, sub)
        _agg_pass(n_oo, row_hbm, col_hbm, poo, z128, acc, gidx, sidx, rows,
                  gsem, ssem, sub)


_s4 = functools.partial(
    pl.kernel, _s4_body,
    out_type=[_mat_sh] * 4,
    mesh=plsc.VectorSubcoreMesh(**_MESH),
    compiler_params=_SC_PARAMS,
    scratch_types=_agg_scratch,
)()


def _t2_body(pa, pt, pio, poi, pii, poo,
             dr_ref, dc_ref, d2a_ref, d2t_ref,
             wsd, wds, w0, w1, w2, w3,
             bsd, bds, b0, b1, b2, b3, o_ref):
    d2a = d2a_ref[...]
    d2t = d2t_ref[...]

    def term(pre_ref, dcol, coeff, w_ref):
        h = (coeff * _rs(dcol)) * pre_ref[...]
        return lax.dot_general(h, w_ref[...], (((1,), (1,)), ((), ())),
                               preferred_element_type=jnp.float32)

    acc = term(pa, dr_ref[:, 0:1], C_A, wsd)
    acc += term(pt, dc_ref[:, 0:1], C_AT, wds)
    acc += term(pio, d2a[:, 16:17], C_IO, w0)
    acc += term(poi, d2t[:, 0:1], C_OI, w1)
    acc += term(pii, d2a[:, 0:1], C_II, w2)
    acc += term(poo, d2t[:, 16:17], C_OO, w3)
    bsum = (C_A * bsd[...] + C_AT * bds[...] + C_IO * b0[...]
            + C_OI * b1[...] + C_II * b2[...] + C_OO * b3[...])
    o_ref[...] = acc + bsum


def _t2(pres, degs, ws, bs):
    pre_spec = pl.BlockSpec((_BT, D), lambda i: (i, 0))
    deg_spec = pl.BlockSpec((_BT, 16), lambda i: (i, 0))
    deg2_spec = pl.BlockSpec((_BT, 32), lambda i: (i, 0))
    w_spec = pl.BlockSpec((D, D), lambda i: (0, 0))
    b_spec = pl.BlockSpec((1, D), lambda i: (0, 0))
    return pl.pallas_call(
        _t2_body,
        grid=(NPAD // _BT,),
        in_specs=([pre_spec] * 6 + [deg_spec, deg_spec, deg2_spec, deg2_spec]
                  + [w_spec] * 6 + [b_spec] * 6),
        out_specs=pl.BlockSpec((_BT, D), lambda i: (i, 0)),
        out_shape=jax.ShapeDtypeStruct((NPAD, D), jnp.float32),
    )(*pres, *degs, *ws, *bs)


def kernel(x, edge_index, W_sd, b_sd, W_ds, b_ds,
           Wx0, bx0, Wx1, bx1, Wx2, bx2, Wx3, bx3):
    row = edge_index[0]
    col = edge_index[1]
    pad = EPAD - row.shape[0]
    sink = jnp.full((pad,), N0, jnp.int32)
    rowp = jnp.concatenate([row.astype(jnp.int32), sink])
    colp = jnp.concatenate([col.astype(jnp.int32), sink])
    xp = jnp.zeros((NPAD, D), jnp.float32).at[:N0].set(x)

    z16 = jnp.zeros((NPAD, 16), jnp.float32)
    z32 = jnp.zeros((NPAD, 32), jnp.float32)
    z128 = jnp.zeros((NPAD, D), jnp.float32)

    dr16, dc16 = _s1(rowp, colp, z16)
    d1cat = jnp.concatenate([dr16, dc16], axis=1)
    d2a, d2t = _s2(rowp, colp, d1cat, z32)
    y1, y2, y3, y4, y5, y6 = _t1(xp, dr16, dc16, d2a, d2t)
    preA, preT, n_oi, n_io, n_ii, n_oo = _s3(
        rowp, colp, y1, y2, y3, y4, y5, y6, z128)
    pio, pii, poi, poo = _s4(rowp, colp, n_io, n_ii, n_oi, n_oo, z128)
    out = _t2((preA, preT, pio, poi, pii, poo),
              (dr16, dc16, d2a, d2t),
              (W_sd, W_ds, Wx0, Wx1, Wx2, Wx3),
              (b_sd.reshape(1, D), b_ds.reshape(1, D), bx0.reshape(1, D),
               bx1.reshape(1, D), bx2.reshape(1, D), bx3.reshape(1, D)))
    return out[:N0]


# speedup vs baseline: 7.1866x; 1.0451x over previous
import functools

import jax
import jax.numpy as jnp
from jax import lax
from jax.experimental import pallas as pl
from jax.experimental.pallas import tpu as pltpu
from jax.experimental.pallas import tpu_sc as plsc

N0 = 10000
D = 128
NPAD = 10112
E0 = 320000
CHUNK = 128
NSUB = 16
NCHUNKS = 160
EPAD = NSUB * NCHUNKS * CHUNK
ROWS = NPAD // NSUB
G = 8
NGRP = NCHUNKS // G

ALPHA = 0.5
BETA = 0.5
GAMA = 0.5
C_A = (1.0 + ALPHA) * ALPHA
C_AT = (1.0 + ALPHA) * (1.0 - ALPHA)
C_IO = (1.0 + BETA) * BETA
C_OI = (1.0 + BETA) * (1.0 - BETA)
C_II = (1.0 + GAMA) * GAMA
C_OO = (1.0 + GAMA) * (1.0 - GAMA)

_MESH = dict(core_axis_name="c", subcore_axis_name="s")
_SC_PARAMS = pltpu.CompilerParams(use_tc_tiling_on_sc=False)


def _ids():
    return lax.axis_index("c"), lax.axis_index("s")


def _s1_body(row_hbm, col_hbm, z16, dr_out, dc_out, deg, idx, ones, sem):
    del sem
    core, sub = _ids()
    unit = jnp.where(
        lax.broadcasted_iota(jnp.int32, (16,), 0) == 0,
        jnp.float32(1.0), jnp.float32(0.0))

    def fill(i, c):
        ones[i, :] = unit
        return c
    lax.fori_loop(0, CHUNK, fill, 0)

    r0 = sub * ROWS
    sl = pl.ds(r0, ROWS)
    pltpu.sync_copy(z16.at[sl], deg.at[sl])
    plsc.subcore_barrier()

    def run(src_hbm):
        def step(i, c):
            base = (sub * NCHUNKS + i) * CHUNK
            pltpu.sync_copy(src_hbm.at[pl.ds(base, CHUNK)], idx)
            pltpu.sync_copy(ones, deg.at[idx], add=True)
            return c
        lax.fori_loop(0, NCHUNKS, step, 0)

    @pl.when(core == 0)
    def _():
        run(row_hbm)

    @pl.when(core == 1)
    def _():
        run(col_hbm)

    plsc.subcore_barrier()

    @pl.when(core == 0)
    def _():
        pltpu.sync_copy(deg.at[sl], dr_out.at[sl])

    @pl.when(core == 1)
    def _():
        pltpu.sync_copy(deg.at[sl], dc_out.at[sl])


_deg_sh = jax.ShapeDtypeStruct((NPAD, 16), jnp.float32)
_s1 = functools.partial(
    pl.kernel, _s1_body,
    out_type=[_deg_sh, _deg_sh],
    mesh=plsc.VectorSubcoreMesh(**_MESH),
    compiler_params=_SC_PARAMS,
    scratch_types=[
        pltpu.VMEM_SHARED((NPAD, 16), jnp.float32),
        pltpu.VMEM((CHUNK,), jnp.int32),
        pltpu.VMEM((CHUNK, 16), jnp.float32),
        pltpu.SemaphoreType.DMA,
    ],
)()


def _s2_body(pidx_a, pidx_t, d1cat, z32, d2a_out, d2t_out,
             acc, pidxb, rows, gs0, gs1, ss0, ss1):
    core, sub = _ids()
    gsem = (gs0, gs1)
    ssem = (ss0, ss1)

    @pl.when(core == 0)
    def _():
        _agg_pass(d1cat, pidx_a, d2a_out, z32, acc, pidxb, rows,
                  gsem, ssem, sub)

    @pl.when(core == 1)
    def _():
        _agg_pass(d1cat, pidx_t, d2t_out, z32, acc, pidxb, rows,
                  gsem, ssem, sub)


_deg2_sh = jax.ShapeDtypeStruct((NPAD, 32), jnp.float32)
_s2 = functools.partial(
    pl.kernel, _s2_body,
    out_type=[_deg2_sh, _deg2_sh],
    mesh=plsc.VectorSubcoreMesh(**_MESH),
    compiler_params=_SC_PARAMS,
    scratch_types=[
        pltpu.VMEM_SHARED((NPAD, 32), jnp.float32),
        pltpu.VMEM((2, 2, CHUNK), jnp.int32),
        pltpu.VMEM((2, CHUNK, 32), jnp.float32),
        pltpu.SemaphoreType.DMA,
        pltpu.SemaphoreType.DMA,
        pltpu.SemaphoreType.DMA,
        pltpu.SemaphoreType.DMA,
    ],
)()


def _rs(d):
    return jnp.where(d > 0, lax.rsqrt(d), 0.0)


def _t1_body(x_ref, dr_ref, dc_ref, d2a_ref, d2t_ref,
             y1_ref, y2_ref, y3_ref, y4_ref, y5_ref, y6_ref):
    xb = x_ref[...]
    d2a = d2a_ref[...]
    d2t = d2t_ref[...]
    y1_ref[...] = _rs(dc_ref[:, 0:1]) * xb
    y2_ref[...] = _rs(dr_ref[:, 0:1]) * xb
    y3_ref[...] = _rs(d2a[:, 16:17]) * xb
    y4_ref[...] = _rs(d2t[:, 0:1]) * xb
    y5_ref[...] = _rs(d2t[:, 16:17]) * xb
    y6_ref[...] = _rs(d2a[:, 0:1]) * xb


_BT = 632


def _t1(xp, dr16, dc16, d2a, d2t):
    y_sh = jax.ShapeDtypeStruct((NPAD, D), jnp.float32)
    x_spec = pl.BlockSpec((_BT, D), lambda i: (i, 0))
    d_spec = pl.BlockSpec((_BT, 16), lambda i: (i, 0))
    d2_spec = pl.BlockSpec((_BT, 32), lambda i: (i, 0))
    return pl.pallas_call(
        _t1_body,
        grid=(NPAD // _BT,),
        in_specs=[x_spec, d_spec, d_spec, d2_spec, d2_spec],
        out_specs=[x_spec] * 6,
        out_shape=[y_sh] * 6,
    )(xp, dr16, dc16, d2a, d2t)


def _agg_pass(y_hbm, pidx_hbm, out_hbm, zeros, acc, pidxb, rows,
              gsem, ssem, sub):
    r0 = sub * ROWS
    sl = pl.ds(r0, ROWS)
    pltpu.sync_copy(zeros.at[sl], acc.at[sl])
    plsc.subcore_barrier()

    def start_gather(i, b):
        base = (sub * NCHUNKS + i) * 2
        pltpu.sync_copy(pidx_hbm.at[pl.ds(base, 2)], pidxb.at[b])
        pltpu.async_copy(y_hbm.at[pidxb.at[b, 0]], rows.at[b], gsem[b])

    def wait_gather(b):
        pltpu.make_async_copy(y_hbm.at[pidxb.at[b, 0]], rows.at[b],
                              gsem[b]).wait()

    def start_scatter(b):
        pltpu.async_copy(rows.at[b], acc.at[pidxb.at[b, 1]], ssem[b],
                         add=True)

    def wait_scatter(b):
        pltpu.make_async_copy(rows.at[b], acc.at[pidxb.at[b, 1]],
                              ssem[b]).wait()

    start_gather(0, 0)
    start_gather(1, 1)
    wait_gather(0)
    start_scatter(0)

    def body(g, c):
        wait_scatter(0)
        start_gather(2 * g + 2, 0)
        wait_gather(1)
        start_scatter(1)
        wait_scatter(1)
        start_gather(2 * g + 3, 1)
        wait_gather(0)
        start_scatter(0)
        return c
    lax.fori_loop(0, (NCHUNKS - 2) // 2, body, 0)
    wait_gather(1)
    start_scatter(1)
    wait_scatter(0)
    wait_scatter(1)
    plsc.subcore_barrier()
    pltpu.sync_copy(acc.at[sl], out_hbm.at[sl])
    plsc.subcore_barrier()


def _s3_body(pidx_a, pidx_t, y1, y2, y3, y4, y5, y6, z128,
             preA, preT, n_oi, n_io, n_ii, n_oo,
             acc, pidxb, rows, gs0, gs1, ss0, ss1):
    core, sub = _ids()
    gsem = (gs0, gs1)
    ssem = (ss0, ss1)

    @pl.when(core == 0)
    def _():
        _agg_pass(y1, pidx_a, preA, z128, acc, pidxb, rows, gsem, ssem, sub)
        _agg_pass(y4, pidx_a, n_oi, z128, acc, pidxb, rows, gsem, ssem, sub)
        _agg_pass(y5, pidx_a, n_ii, z128, acc, pidxb, rows, gsem, ssem, sub)

    @pl.when(core == 1)
    def _():
        _agg_pass(y2, pidx_t, preT, z128, acc, pidxb, rows, gsem, ssem, sub)
        _agg_pass(y3, pidx_t, n_io, z128, acc, pidxb, rows, gsem, ssem, sub)
        _agg_pass(y6, pidx_t, n_oo, z128, acc, pidxb, rows, gsem, ssem, sub)


_mat_sh = jax.ShapeDtypeStruct((NPAD, D), jnp.float32)
_agg_scratch = [
    pltpu.VMEM_SHARED((NPAD, D), jnp.float32),
    pltpu.VMEM((2, 2, CHUNK), jnp.int32),
    pltpu.VMEM((2, CHUNK, D), jnp.float32),
    pltpu.SemaphoreType.DMA,
    pltpu.SemaphoreType.DMA,
    pltpu.SemaphoreType.DMA,
    pltpu.SemaphoreType.DMA,
]

_s3 = functools.partial(
    pl.kernel, _s3_body,
    out_type=[_mat_sh] * 6,
    mesh=plsc.VectorSubcoreMesh(**_MESH),
    compiler_params=_SC_PARAMS,
    scratch_types=_agg_scratch,
)()


def _s4_body(pidx_a, pidx_t, n_io, n_ii, n_oi, n_oo, z128,
             pio, pii, poi, poo,
             acc, pidxb, rows, gs0, gs1, ss0, ss1):
    core, sub = _ids()
    gsem = (gs0, gs1)
    ssem = (ss0, ss1)

    @pl.when(core == 0)
    def _():
        _agg_pass(n_io, pidx_a, pio, z128, acc, pidxb, rows, gsem, ssem, sub)
        _agg_pass(n_ii, pidx_a, pii, z128, acc, pidxb, rows, gsem, ssem, sub)

    @pl.when(core == 1)
    def _():
        _agg_pass(n_oi, pidx_t, poi, z128, acc, pidxb, rows, gsem, ssem, sub)
        _agg_pass(n_oo, pidx_t, poo, z128, acc, pidxb, rows, gsem, ssem, sub)


_s4 = functools.partial(
    pl.kernel, _s4_body,
    out_type=[_mat_sh] * 4,
    mesh=plsc.VectorSubcoreMesh(**_MESH),
    compiler_params=_SC_PARAMS,
    scratch_types=_agg_scratch,
)()


def _t2_body(pa, pt, pio, poi, pii, poo,
             dr_ref, dc_ref, d2a_ref, d2t_ref,
             wsd, wds, w0, w1, w2, w3,
             bsd, bds, b0, b1, b2, b3, o_ref):
    d2a = d2a_ref[...]
    d2t = d2t_ref[...]

    def term(pre_ref, dcol, coeff, w_ref):
        h = (coeff * _rs(dcol)) * pre_ref[...]
        return lax.dot_general(h, w_ref[...], (((1,), (1,)), ((), ())),
                               preferred_element_type=jnp.float32)

    acc = term(pa, dr_ref[:, 0:1], C_A, wsd)
    acc += term(pt, dc_ref[:, 0:1], C_AT, wds)
    acc += term(pio, d2a[:, 16:17], C_IO, w0)
    acc += term(poi, d2t[:, 0:1], C_OI, w1)
    acc += term(pii, d2a[:, 0:1], C_II, w2)
    acc += term(poo, d2t[:, 16:17], C_OO, w3)
    bsum = (C_A * bsd[...] + C_AT * bds[...] + C_IO * b0[...]
            + C_OI * b1[...] + C_II * b2[...] + C_OO * b3[...])
    o_ref[...] = acc + bsum


def _t2(pres, degs, ws, bs):
    pre_spec = pl.BlockSpec((_BT, D), lambda i: (i, 0))
    deg_spec = pl.BlockSpec((_BT, 16), lambda i: (i, 0))
    deg2_spec = pl.BlockSpec((_BT, 32), lambda i: (i, 0))
    w_spec = pl.BlockSpec((D, D), lambda i: (0, 0))
    b_spec = pl.BlockSpec((1, D), lambda i: (0, 0))
    return pl.pallas_call(
        _t2_body,
        grid=(NPAD // _BT,),
        in_specs=([pre_spec] * 6 + [deg_spec, deg_spec, deg2_spec, deg2_spec]
                  + [w_spec] * 6 + [b_spec] * 6),
        out_specs=pl.BlockSpec((_BT, D), lambda i: (i, 0)),
        out_shape=jax.ShapeDtypeStruct((NPAD, D), jnp.float32),
    )(*pres, *degs, *ws, *bs)


def kernel(x, edge_index, W_sd, b_sd, W_ds, b_ds,
           Wx0, bx0, Wx1, bx1, Wx2, bx2, Wx3, bx3):
    row = edge_index[0]
    col = edge_index[1]
    pad = EPAD - row.shape[0]
    sink = jnp.full((pad,), N0, jnp.int32)
    rowp = jnp.concatenate([row.astype(jnp.int32), sink])
    colp = jnp.concatenate([col.astype(jnp.int32), sink])
    xp = jnp.zeros((NPAD, D), jnp.float32).at[:N0].set(x)

    z16 = jnp.zeros((NPAD, 16), jnp.float32)
    z32 = jnp.zeros((NPAD, 32), jnp.float32)
    z128 = jnp.zeros((NPAD, D), jnp.float32)

    row2 = rowp.reshape(-1, CHUNK)
    col2 = colp.reshape(-1, CHUNK)
    pidx_a = jnp.stack([col2, row2], axis=1).reshape(-1, CHUNK)
    pidx_t = jnp.stack([row2, col2], axis=1).reshape(-1, CHUNK)

    dr16, dc16 = _s1(rowp, colp, z16)
    d1cat = jnp.concatenate([dr16, dc16], axis=1)
    d2a, d2t = _s2(pidx_a, pidx_t, d1cat, z32)
    y1, y2, y3, y4, y5, y6 = _t1(xp, dr16, dc16, d2a, d2t)
    preA, preT, n_oi, n_io, n_ii, n_oo = _s3(
        pidx_a, pidx_t, y1, y2, y3, y4, y5, y6, z128)
    pio, pii, poi, poo = _s4(pidx_a, pidx_t, n_io, n_ii, n_oi, n_oo, z128)
    out = _t2((preA, preT, pio, poi, pii, poo),
              (dr16, dc16, d2a, d2t),
              (W_sd, W_ds, Wx0, Wx1, Wx2, Wx3),
              (b_sd.reshape(1, D), b_ds.reshape(1, D), bx0.reshape(1, D),
               bx1.reshape(1, D), bx2.reshape(1, D), bx3.reshape(1, D)))
    return out[:N0]
